# Initial kernel scaffold; baseline (speedup 1.0000x reference)
import jax
import jax.numpy as jnp
from jax.experimental import pallas as pl


def kernel(x, seg, tok_table, pos_table, seg_table, gamma, beta):
    raise NotImplementedError("write your pallas kernel here")


# speedup vs baseline: 1.5972x; 1.5972x over previous
import functools

import jax
import jax.numpy as jnp
import numpy as np
from jax import lax
from jax.experimental import pallas as pl
from jax.experimental.pallas import tpu as pltpu
from jax.experimental.pallas import tpu_sc as plsc

D = 128
L = 16
NV = D // L
CHUNK = 128
MAXLEN = 200

_GDN = lax.GatherDimensionNumbers(
    offset_dims=(), collapsed_slice_dims=(0,), start_index_map=(0,))


def _bcast_lane(v, k):
  idx = jnp.full((L, 1), k, dtype=jnp.int32)
  return lax.gather(v, idx, _GDN, (1,),
                    mode=lax.GatherScatterMode.PROMISE_IN_BOUNDS)


def _perm(v, idx):
  return lax.gather(v, lax.reshape(idx, (L, 1)), _GDN, (1,),
                    mode=lax.GatherScatterMode.PROMISE_IN_BOUNDS)


def _bfly_sum(v, iota):
  for m in (1, 2, 4, 8):
    v = v + _perm(v, iota ^ m)
  return v


def _rsqrt_newton(v):
  i = lax.bitcast_convert_type(v, jnp.int32)
  i = jnp.int32(0x5F3759DF) - lax.shift_right_logical(i, 1)
  y = lax.bitcast_convert_type(i, jnp.float32)
  vh = v * 0.5
  for _ in range(3):
    y = y * (1.5 - vh * y * y)
  return y


def _sc_embed_ln(n_rows, n_workers):
  rows_per_w = n_rows // n_workers
  n_chunks = rows_per_w // CHUNK
  mesh = plsc.VectorSubcoreMesh(core_axis_name="c", subcore_axis_name="s")

  @functools.partial(
      pl.kernel,
      out_type=jax.ShapeDtypeStruct((n_rows, D), jnp.float32),
      mesh=mesh,
      scratch_types=dict(
          idx_v=pltpu.VMEM((CHUNK,), jnp.int32),
          seg_v=pltpu.VMEM((CHUNK,), jnp.int32),
          rows_v=pltpu.VMEM((CHUNK, D), jnp.float32),
          pos_v=pltpu.VMEM((MAXLEN * D,), jnp.float32),
          dvec_v=pltpu.VMEM((D,), jnp.float32),
          gam_v=pltpu.VMEM((D,), jnp.float32),
          bet_v=pltpu.VMEM((D,), jnp.float32),
          sem=pltpu.SemaphoreType.DMA,
      ),
  )
  def k(x_hbm, seg_hbm, tok_hbm, pos2_hbm, dvec_hbm, gam_hbm, bet_hbm,
        out_hbm, idx_v, seg_v, rows_v, pos_v, dvec_v, gam_v, bet_v, sem):
    wid = lax.axis_index("c") * 16 + lax.axis_index("s")
    w_base = wid * rows_per_w

    pltpu.sync_copy(pos2_hbm, pos_v)
    pltpu.sync_copy(dvec_hbm, dvec_v)
    pltpu.sync_copy(gam_hbm, gam_v)
    pltpu.sync_copy(bet_hbm, bet_v)

    iota = lax.iota(jnp.int32, L)
    inv_d = jnp.float32(1.0 / D)
    eps = jnp.float32(1e-5)

    @pl.loop(0, n_chunks)
    def _chunk(t):
      base = w_base + t * CHUNK
      pltpu.sync_copy(x_hbm.at[pl.ds(base, CHUNK)], idx_v)
      pltpu.sync_copy(seg_hbm.at[pl.ds(base, CHUNK)], seg_v)
      pltpu.async_copy(tok_hbm.at[idx_v], rows_v, sem).wait()

      @pl.loop(0, CHUNK // L)
      def _group(g):
        segf = seg_v[pl.ds(g * L, L)].astype(jnp.float32)
        s_acc = jnp.zeros((L,), jnp.float32)
        q_acc = jnp.zeros((L,), jnp.float32)
        for kk in range(L):
          r = g * L + kk
          p = lax.rem(base + r, MAXLEN)
          sb = _bcast_lane(segf, kk)
          tvals = []
          for i in range(NV):
            tv = (rows_v[r, pl.ds(i * L, L)]
                  + pos_v[pl.ds(p * D + i * L, L)]
                  + sb * dvec_v[pl.ds(i * L, L)])
            tvals.append(tv)
            rows_v[r, pl.ds(i * L, L)] = tv
          s = tvals[0]
          q = tvals[0] * tvals[0]
          for i in range(1, NV):
            s = s + tvals[i]
            q = q + tvals[i] * tvals[i]
          s_acc = jnp.where(iota == kk, _bfly_sum(s, iota), s_acc)
          q_acc = jnp.where(iota == kk, _bfly_sum(q, iota), q_acc)
        mean = s_acc * inv_d
        var = q_acc * inv_d - mean * mean
        rstd = _rsqrt_newton(var + eps)
        shift = -(mean * rstd)
        for kk in range(L):
          r = g * L + kk
          a = _bcast_lane(rstd, kk)
          b = _bcast_lane(shift, kk)
          for i in range(NV):
            tv = rows_v[r, pl.ds(i * L, L)]
            rows_v[r, pl.ds(i * L, L)] = (
                (tv * a + b) * gam_v[pl.ds(i * L, L)] + bet_v[pl.ds(i * L, L)])

      pltpu.sync_copy(rows_v, out_hbm.at[pl.ds(base, CHUNK)])

  return k


@jax.jit
def kernel(x, seg, tok_table, pos_table, seg_table, gamma, beta):
  bsz, seq = x.shape
  n_rows = bsz * seq
  x_flat = x.reshape(n_rows).astype(jnp.int32)
  seg_flat = seg.reshape(n_rows).astype(jnp.int32)
  pos2 = (pos_table + seg_table[0][None, :]).astype(jnp.float32).reshape(-1)
  dvec = (seg_table[1] - seg_table[0]).astype(jnp.float32)
  info = plsc.get_sparse_core_info()
  n_workers = info.num_cores * info.num_subcores
  out = _sc_embed_ln(n_rows, n_workers)(
      x_flat, seg_flat, tok_table.astype(jnp.float32), pos2, dvec,
      gamma.astype(jnp.float32), beta.astype(jnp.float32))
  return out.reshape(bsz, seq, D)
